# baseline (speedup 1.0000x reference)
import functools

import jax
import jax.numpy as jnp
from jax.experimental import pallas as pl
from jax.experimental.pallas import tpu as pltpu


def _se_kernel(x_ref, w1_ref, b1_ref, w2_ref, b2_ref, o_ref, *, inv_hw):
    xb = x_ref[0]
    s = jnp.sum(xb, axis=1, keepdims=True) * inv_hw
    h = jnp.dot(w1_ref[...], s, preferred_element_type=jnp.float32)
    h = jnp.maximum(h + b1_ref[...], 0.0)
    g = jnp.dot(w2_ref[...], h, preferred_element_type=jnp.float32)
    g = jax.nn.sigmoid(g + b2_ref[...])
    o_ref[0] = xb * g


def kernel(x, w1, b1, w2, b2):
    B, C, H, W = x.shape
    S = H * W
    BOT = w1.shape[0]
    xr = x.reshape(B, C, S)
    b1c = b1.reshape(BOT, 1)
    b2c = b2.reshape(C, 1)

    body = functools.partial(_se_kernel, inv_hw=1.0 / S)

    out = pl.pallas_call(
        body,
        grid=(B,),
        in_specs=[
            pl.BlockSpec((1, C, S), lambda b: (b, 0, 0)),
            pl.BlockSpec((BOT, C), lambda b: (0, 0)),
            pl.BlockSpec((BOT, 1), lambda b: (0, 0)),
            pl.BlockSpec((C, BOT), lambda b: (0, 0)),
            pl.BlockSpec((C, 1), lambda b: (0, 0)),
        ],
        out_specs=pl.BlockSpec((1, C, S), lambda b: (b, 0, 0)),
        out_shape=jax.ShapeDtypeStruct((B, C, S), jnp.float32),
        compiler_params=pltpu.CompilerParams(
            dimension_semantics=("parallel",),
        ),
    )(xr, w1, b1c, w2, b2c)
    return out.reshape(B, C, H, W)
